# Initial kernel scaffold; baseline (speedup 1.0000x reference)
import functools

import jax
import jax.numpy as jnp
from jax import lax
from jax.experimental import pallas as pl
from jax.experimental.pallas import tpu as pltpu
from jax.experimental.pallas import tpu_sc as plsc

N = 320000
D = 128
S = 10000


BLK = 6400


def _k1_body(x_ref, w_ref, b_ref, lg_ref, m_ref):
    i = pl.program_id(0)
    x = x_ref[...]
    wv = w_ref[...]
    lg = lax.dot_general(x, wv, (((1,), (1,)), ((), ())),
                         preferred_element_type=jnp.float32)
    lg = lg + b_ref[0, 0]
    lg2 = lg.reshape(BLK // 128, 128)
    lg_ref[...] = lg2

    @pl.when(i == 0)
    def _():
        m_ref[0, 0] = -jnp.inf

    m_ref[0, 0] = jnp.maximum(m_ref[0, 0], jnp.max(lg2))


def _k1(x, W, b):
    nsteps = N // BLK
    return pl.pallas_call(
        _k1_body,
        grid=(nsteps,),
        in_specs=[
            pl.BlockSpec((BLK, D), lambda i: (i, 0)),
            pl.BlockSpec((1, D), lambda i: (0, 0)),
            pl.BlockSpec((1, 1), lambda i: (0, 0)),
        ],
        out_specs=[
            pl.BlockSpec((BLK // 128, 128), lambda i: (i, 0)),
            pl.BlockSpec((1, 1), lambda i: (0, 0)),
        ],
        out_shape=[
            jax.ShapeDtypeStruct((N // 128, 128), jnp.float32),
            jax.ShapeDtypeStruct((1, 1), jnp.float32),
        ],
    )(x, W, b)


def _k1b_body(lg_ref, m_ref, w_ref):
    e = jnp.exp(lg_ref[...] - m_ref[0, 0])
    w_ref[...] = e / jnp.sum(e)


def _k1b(lg2d, m):
    return pl.pallas_call(
        _k1b_body,
        out_shape=jax.ShapeDtypeStruct((N // 128, 128), jnp.float32),
    )(lg2d, m)


NC = 2
NS = 16
RPT = N // (NC * NS)
C = 400
NCHUNK = RPT // C
ZR = S // NS


def _k2_body(x_hbm, w_hbm, lab_hbm, out_hbm, xb, wb, lb, acc):
    c = lax.axis_index("c")
    s = lax.axis_index("s")
    base = (c * NS + s) * RPT

    def zbody(r, _):
        for j in range(D // 16):
            xb[r, pl.ds(j * 16, 16)] = jnp.zeros((16,), jnp.float32)
        return 0

    lax.fori_loop(0, C, zbody, 0)
    pltpu.sync_copy(xb, acc.at[pl.ds(s * ZR, C)])
    pltpu.sync_copy(xb.at[pl.ds(0, ZR - C)], acc.at[pl.ds(s * ZR + C, ZR - C)])
    plsc.subcore_barrier()

    def chunk(g, _):
        off = base + g * C
        pltpu.sync_copy(x_hbm.at[pl.ds(off, C)], xb)
        pltpu.sync_copy(w_hbm.at[pl.ds(off, C)], wb)
        pltpu.sync_copy(lab_hbm.at[pl.ds(off, C)], lb)

        def wbody(r, _):
            wv = jnp.full((16,), wb[r], jnp.float32)
            for j in range(D // 16):
                sl = pl.ds(j * 16, 16)
                xb[r, sl] = xb[r, sl] * wv
            return 0

        lax.fori_loop(0, C, wbody, 0)
        pltpu.sync_copy(xb, acc.at[lb], add=True)
        return 0

    lax.fori_loop(0, NCHUNK, chunk, 0)
    plsc.subcore_barrier()

    pltpu.sync_copy(acc.at[pl.ds(s * ZR, C)], xb)
    pltpu.sync_copy(xb, out_hbm.at[c, pl.ds(s * ZR, C)])
    pltpu.sync_copy(acc.at[pl.ds(s * ZR + C, ZR - C)], xb.at[pl.ds(0, ZR - C)])
    pltpu.sync_copy(xb.at[pl.ds(0, ZR - C)], out_hbm.at[c, pl.ds(s * ZR + C, ZR - C)])


_k2 = functools.partial(
    pl.kernel,
    _k2_body,
    out_type=jax.ShapeDtypeStruct((NC, S, D), jnp.float32),
    mesh=plsc.VectorSubcoreMesh(core_axis_name="c", subcore_axis_name="s"),
    scratch_types=[
        pltpu.VMEM((C, D), jnp.float32),
        pltpu.VMEM((C,), jnp.float32),
        pltpu.VMEM((C,), jnp.int32),
        pltpu.VMEM_SHARED((S, D), jnp.float32),
    ],
)()


K3B = 1000


def _k3_body(h_ref, o_ref):
    o_ref[...] = h_ref[0] + h_ref[1]


def _k3(halves):
    return pl.pallas_call(
        _k3_body,
        grid=(S // K3B,),
        in_specs=[pl.BlockSpec((NC, K3B, D), lambda i: (0, i, 0))],
        out_specs=pl.BlockSpec((K3B, D), lambda i: (i, 0)),
        out_shape=jax.ShapeDtypeStruct((S, D), jnp.float32),
    )(halves)


def kernel(x, monomer_labels_i, W, b):
    lg2d, m = _k1(x, W, b.reshape(1, 1))
    w2d = _k1b(lg2d, m)
    halves = _k2(x, w2d.reshape(N), monomer_labels_i)
    return _k3(halves)


# speedup vs baseline: 3.2601x; 3.2601x over previous
import functools

import jax
import jax.numpy as jnp
from jax import lax
from jax.experimental import pallas as pl
from jax.experimental.pallas import tpu as pltpu
from jax.experimental.pallas import tpu_sc as plsc

N = 320000
D = 128
S = 10000


BLK = 3200


def _k1_body(x_ref, w_ref, b_ref, lg_ref, m_ref):
    i = pl.program_id(0)
    x = x_ref[...]
    wv = w_ref[...]
    lg = lax.dot_general(wv, x, (((1,), (1,)), ((), ())),
                         preferred_element_type=jnp.float32)
    lg = lg + b_ref[0, 0]
    lg_ref[...] = lg

    @pl.when(i == 0)
    def _():
        m_ref[0, 0] = -jnp.inf

    m_ref[0, 0] = jnp.maximum(m_ref[0, 0], jnp.max(lg))


def _k1(x, W, b):
    nsteps = N // BLK
    return pl.pallas_call(
        _k1_body,
        grid=(nsteps,),
        in_specs=[
            pl.BlockSpec((BLK, D), lambda i: (i, 0)),
            pl.BlockSpec((1, D), lambda i: (0, 0)),
            pl.BlockSpec((1, 1), lambda i: (0, 0)),
        ],
        out_specs=[
            pl.BlockSpec((1, BLK), lambda i: (0, i)),
            pl.BlockSpec(memory_space=pltpu.MemorySpace.SMEM),
        ],
        out_shape=[
            jax.ShapeDtypeStruct((1, N), jnp.float32),
            jax.ShapeDtypeStruct((1, 1), jnp.float32),
        ],
    )(x, W, b)


def _k1b_body(lg_ref, m_ref, w_ref):
    e = jnp.exp(lg_ref[...] - m_ref[0, 0])
    w_ref[...] = e / jnp.sum(e)


def _k1b(lg2d, m):
    return pl.pallas_call(
        _k1b_body,
        in_specs=[
            pl.BlockSpec((N // 128, 128), lambda: (0, 0)),
            pl.BlockSpec(memory_space=pltpu.MemorySpace.SMEM),
        ],
        out_shape=jax.ShapeDtypeStruct((N // 128, 128), jnp.float32),
    )(lg2d, m)


NC = 2
NS = 16
RPT = N // (NC * NS)
C = 200
NCHUNK = RPT // C
ZR = 624


def _acc_copy_plan(total):
    plan, off = [], 0
    while off < total:
        n = min(C, total - off)
        plan.append((off, n))
        off += n
    return plan


def _k2_body(x_hbm, w_hbm, lab_hbm, out_hbm, xb, wb, lb, acc):
    c = lax.axis_index("c")
    s = lax.axis_index("s")
    base = (c * NS + s) * RPT

    def zbody(r, _):
        for j in range(D // 16):
            xb[r, pl.ds(j * 16, 16)] = jnp.zeros((16,), jnp.float32)
        return 0

    lax.fori_loop(0, C, zbody, 0)
    for off, n in _acc_copy_plan(ZR):
        pltpu.sync_copy(xb.at[pl.ds(0, n)], acc.at[pl.ds(s * ZR + off, n)])

    @pl.when(s == NS - 1)
    def _():
        pltpu.sync_copy(xb.at[pl.ds(0, S - NS * ZR)],
                        acc.at[pl.ds(NS * ZR, S - NS * ZR)])

    plsc.subcore_barrier()

    def chunk(g, _):
        off = base + g * C
        pltpu.sync_copy(x_hbm.at[pl.ds(off, C)], xb)
        pltpu.sync_copy(w_hbm.at[pl.ds(off, C)], wb)
        pltpu.sync_copy(lab_hbm.at[pl.ds(off, C)], lb)

        def wbody(g16, _):
            wv16 = wb[pl.ds(g16 * 16, 16)]
            for k in range(16):
                bw = jnp.full((16,), wv16[k], jnp.float32)
                row = g16 * 16 + k
                for j in range(D // 16):
                    sl = pl.ds(j * 16, 16)
                    xb[row, sl] = xb[row, sl] * bw
            return 0

        lax.fori_loop(0, C // 16, wbody, 0)
        if C % 16:
            off16 = C - 16
            wv16 = wb[pl.ds(off16, 16)]
            for k in range(16 - C % 16, 16):
                bw = jnp.full((16,), wv16[k], jnp.float32)
                row = off16 + k
                for j in range(D // 16):
                    sl = pl.ds(j * 16, 16)
                    xb[row, sl] = xb[row, sl] * bw
        pltpu.sync_copy(xb, acc.at[lb], add=True)
        return 0

    lax.fori_loop(0, NCHUNK, chunk, 0)
    plsc.subcore_barrier()

    for off, n in _acc_copy_plan(ZR):
        pltpu.sync_copy(acc.at[pl.ds(s * ZR + off, n)], xb.at[pl.ds(0, n)])
        pltpu.sync_copy(xb.at[pl.ds(0, n)], out_hbm.at[c, pl.ds(s * ZR + off, n)])

    @pl.when(s == NS - 1)
    def _():
        rem = S - NS * ZR
        pltpu.sync_copy(acc.at[pl.ds(NS * ZR, rem)], xb.at[pl.ds(0, rem)])
        pltpu.sync_copy(xb.at[pl.ds(0, rem)], out_hbm.at[c, pl.ds(NS * ZR, rem)])


_k2 = pl.kernel(
    _k2_body,
    out_type=jax.ShapeDtypeStruct((NC, S, D), jnp.float32),
    mesh=plsc.VectorSubcoreMesh(core_axis_name="c", subcore_axis_name="s"),
    scratch_types=[
        pltpu.VMEM((C, D), jnp.float32),
        pltpu.VMEM((C,), jnp.float32),
        pltpu.VMEM((C,), jnp.int32),
        pltpu.VMEM_SHARED((S, D), jnp.float32),
    ],
)


K3B = 1000


def _k3_body(h_ref, o_ref):
    o_ref[...] = h_ref[0] + h_ref[1]


def _k3(halves):
    return pl.pallas_call(
        _k3_body,
        grid=(S // K3B,),
        in_specs=[pl.BlockSpec((NC, K3B, D), lambda i: (0, i, 0))],
        out_specs=pl.BlockSpec((K3B, D), lambda i: (i, 0)),
        out_shape=jax.ShapeDtypeStruct((S, D), jnp.float32),
    )(halves)


def kernel(x, monomer_labels_i, W, b):
    lg, m = _k1(x, W, b.reshape(1, 1))
    w2d = _k1b(lg.reshape(N // 128, 128), m)
    halves = _k2(x, w2d.reshape(N), monomer_labels_i)
    return _k3(halves)
